# Initial kernel scaffold; baseline (speedup 1.0000x reference)
import jax
import jax.numpy as jnp
from jax.experimental import pallas as pl


def kernel(x, const):
    raise NotImplementedError("write your pallas kernel here")


# speedup vs baseline: 1.3192x; 1.3192x over previous
import dataclasses
import functools

import jax
import jax.numpy as jnp
from jax import lax
from jax.experimental import pallas as pl
from jax.experimental.pallas import tpu as pltpu
from jax.experimental.pallas import tpu_sc as plsc

_H = 384
_W = 384
_N = _H * _W
_BC = 384
_NC = 2
_NS = 16
_NW = _NC * _NS
_PER_W = _N // _NW
_G = 32
_NSTEP = _PER_W // _G
_L = 16


def _sc_interp(table, idx4, wt4):
    mesh = plsc.VectorSubcoreMesh(core_axis_name="c", subcore_axis_name="s")
    cp = pltpu.CompilerParams()
    if "needs_layout_passes" in pltpu.CompilerParams.__dataclass_fields__:
        cp = dataclasses.replace(cp, needs_layout_passes=False)

    @functools.partial(
        pl.kernel,
        mesh=mesh,
        compiler_params=cp,
        out_type=jax.ShapeDtypeStruct((_N, _BC), jnp.float32),
        scratch_types=(
            [pltpu.VMEM((_PER_W,), jnp.int32) for _ in range(4)]
            + [pltpu.VMEM((_PER_W,), jnp.float32) for _ in range(4)]
            + [
                pltpu.VMEM((4, _G, _BC), jnp.float32),
                pltpu.VMEM((_G, _BC), jnp.float32),
                pltpu.SemaphoreType.DMA,
            ]
        ),
    )
    def k(table_h, idx_h, wt_h, out_h,
          i0_v, i1_v, i2_v, i3_v, w0_v, w1_v, w2_v, w3_v, g, outb, sem):
        idx_vs = [i0_v, i1_v, i2_v, i3_v]
        w_vs = [w0_v, w1_v, w2_v, w3_v]
        wid = lax.axis_index("s") * _NC + lax.axis_index("c")
        base = wid * _PER_W
        for kk in range(4):
            pltpu.sync_copy(idx_h.at[kk, pl.ds(base, _PER_W)], idx_vs[kk])
            pltpu.sync_copy(wt_h.at[kk, pl.ds(base, _PER_W)], w_vs[kk])

        @pl.loop(0, _NSTEP)
        def _(c):
            off = c * _G
            cps = [
                pltpu.async_copy(table_h.at[idx_vs[kk].at[pl.ds(off, _G)]],
                                 g.at[kk], sem)
                for kk in range(4)
            ]
            for cp in cps:
                cp.wait()

            @pl.loop(0, _G)
            def _(r):
                ridx = jnp.full((_L,), off + r, jnp.int32)
                ws = [plsc.load_gather(w_vs[kk], [ridx]) for kk in range(4)]
                for j in range(_BC // _L):
                    s = pl.ds(j * _L, _L)
                    acc = ws[0] * g[0, r, s]
                    acc = acc + ws[1] * g[1, r, s]
                    acc = acc + ws[2] * g[2, r, s]
                    acc = acc + ws[3] * g[3, r, s]
                    outb[r, s] = acc

            pltpu.sync_copy(outb, out_h.at[pl.ds(base + off, _G)])

    return k(table, idx4, wt4)


def _indices_weights(theta):
    cy = (_H - 1) / 2.0
    cx = (_W - 1) / 2.0
    gy, gx = jnp.meshgrid(
        jnp.arange(_H, dtype=jnp.float32) - cy,
        jnp.arange(_W, dtype=jnp.float32) - cx,
        indexing="ij",
    )
    cos_t = jnp.cos(theta)
    sin_t = jnp.sin(theta)
    src_x = cos_t * gx + sin_t * gy + cx
    src_y = -sin_t * gx + cos_t * gy + cy
    x0 = jnp.floor(src_x)
    y0 = jnp.floor(src_y)
    wx1 = src_x - x0
    wx0 = 1.0 - wx1
    wy1 = src_y - y0
    wy0 = 1.0 - wy1
    valid = ((src_x >= 0) & (src_x <= _W - 1)
             & (src_y >= 0) & (src_y <= _H - 1)).astype(jnp.float32)
    x0i = jnp.clip(x0, 0, _W - 1).astype(jnp.int32)
    x1i = jnp.clip(x0 + 1.0, 0, _W - 1).astype(jnp.int32)
    y0i = jnp.clip(y0, 0, _H - 1).astype(jnp.int32)
    y1i = jnp.clip(y0 + 1.0, 0, _H - 1).astype(jnp.int32)
    idx4 = jnp.stack([
        (y0i * _W + x0i).reshape(-1),
        (y0i * _W + x1i).reshape(-1),
        (y1i * _W + x0i).reshape(-1),
        (y1i * _W + x1i).reshape(-1),
    ])
    wt4 = jnp.stack([
        (wy0 * wx0 * valid).reshape(-1),
        (wy0 * wx1 * valid).reshape(-1),
        (wy1 * wx0 * valid).reshape(-1),
        (wy1 * wx1 * valid).reshape(-1),
    ])
    return idx4, wt4


def kernel(x, const):
    theta = jnp.squeeze(const, axis=0)[0]
    idx4, wt4 = _indices_weights(theta)
    table = x.reshape(_BC, _N).T
    out_t = _sc_interp(table, idx4, wt4)
    return out_t.T.reshape(x.shape)


# speedup vs baseline: 1.6495x; 1.2504x over previous
import dataclasses
import functools

import jax
import jax.numpy as jnp
from jax import lax
from jax.experimental import pallas as pl
from jax.experimental.pallas import tpu as pltpu
from jax.experimental.pallas import tpu_sc as plsc

_H = 384
_W = 384
_N = _H * _W
_BC = 384
_NC = 2
_NS = 16
_NW = _NC * _NS
_PER_W = _N // _NW
_G = 16
_NSTEP = _PER_W // _G
_L = 16


def _sc_interp(table, idx4, wt4):
    mesh = plsc.VectorSubcoreMesh(core_axis_name="c", subcore_axis_name="s")
    cp = pltpu.CompilerParams()
    if "needs_layout_passes" in pltpu.CompilerParams.__dataclass_fields__:
        cp = dataclasses.replace(cp, needs_layout_passes=False)

    @functools.partial(
        pl.kernel,
        mesh=mesh,
        compiler_params=cp,
        out_type=jax.ShapeDtypeStruct((_N, _BC), jnp.float32),
        scratch_types=(
            [pltpu.VMEM((_PER_W,), jnp.int32) for _ in range(4)]
            + [pltpu.VMEM((_PER_W,), jnp.float32) for _ in range(4)]
            + [
                pltpu.VMEM((2, 4, _G, _BC), jnp.float32),
                pltpu.VMEM((2, _G, _BC), jnp.float32),
                pltpu.SemaphoreType.DMA,
                pltpu.SemaphoreType.DMA,
                pltpu.SemaphoreType.DMA,
                pltpu.SemaphoreType.DMA,
            ]
        ),
    )
    def k(table_h, idx_h, wt_h, out_h,
          i0_v, i1_v, i2_v, i3_v, w0_v, w1_v, w2_v, w3_v, g, outb,
          gsem0, gsem1, osem0, osem1):
        idx_vs = [i0_v, i1_v, i2_v, i3_v]
        w_vs = [w0_v, w1_v, w2_v, w3_v]
        gsems = [gsem0, gsem1]
        osems = [osem0, osem1]
        wid = lax.axis_index("s") * _NC + lax.axis_index("c")
        base = wid * _PER_W

        for kk in range(4):
            pltpu.sync_copy(idx_h.at[kk, pl.ds(base, _PER_W)], idx_vs[kk])
            pltpu.sync_copy(wt_h.at[kk, pl.ds(base, _PER_W)], w_vs[kk])

        def fire_gathers(chunk, par):
            off = chunk * _G
            for kk in range(4):
                pltpu.async_copy(table_h.at[idx_vs[kk].at[pl.ds(off, _G)]],
                                 g.at[par, kk], gsems[par])

        def drain_gathers(chunk, par):
            off = chunk * _G
            for kk in range(4):
                pltpu.make_async_copy(
                    table_h.at[idx_vs[kk].at[pl.ds(off, _G)]],
                    g.at[par, kk], gsems[par]).wait()

        def fire_out(chunk, par):
            off = chunk * _G
            pltpu.async_copy(outb.at[par], out_h.at[pl.ds(base + off, _G)],
                             osems[par])

        def drain_out(par):
            pltpu.make_async_copy(outb.at[par], out_h.at[pl.ds(base, _G)],
                                  osems[par]).wait()

        def compute(chunk, par):
            off = chunk * _G

            @pl.loop(0, _G)
            def _(r):
                ridx = jnp.full((_L,), off + r, jnp.int32)
                ws = [plsc.load_gather(w_vs[kk], [ridx]) for kk in range(4)]
                for j in range(_BC // _L):
                    s = pl.ds(j * _L, _L)
                    acc = ws[0] * g[par, 0, r, s]
                    acc = acc + ws[1] * g[par, 1, r, s]
                    acc = acc + ws[2] * g[par, 2, r, s]
                    acc = acc + ws[3] * g[par, 3, r, s]
                    outb[par, r, s] = acc

        fire_gathers(0, 0)

        @pl.loop(0, _NSTEP, step=2)
        def _(c0):
            for par in range(2):
                chunk = c0 + par

                @pl.when(chunk + 1 < _NSTEP)
                def _():
                    fire_gathers(chunk + 1, 1 - par)

                drain_gathers(chunk, par)

                @pl.when(chunk >= 2)
                def _():
                    drain_out(par)

                compute(chunk, par)
                fire_out(chunk, par)

        drain_out(0)
        drain_out(1)

    return k(table, idx4, wt4)


def _indices_weights(theta):
    cy = (_H - 1) / 2.0
    cx = (_W - 1) / 2.0
    gy, gx = jnp.meshgrid(
        jnp.arange(_H, dtype=jnp.float32) - cy,
        jnp.arange(_W, dtype=jnp.float32) - cx,
        indexing="ij",
    )
    cos_t = jnp.cos(theta)
    sin_t = jnp.sin(theta)
    src_x = cos_t * gx + sin_t * gy + cx
    src_y = -sin_t * gx + cos_t * gy + cy
    x0 = jnp.floor(src_x)
    y0 = jnp.floor(src_y)
    wx1 = src_x - x0
    wx0 = 1.0 - wx1
    wy1 = src_y - y0
    wy0 = 1.0 - wy1
    valid = ((src_x >= 0) & (src_x <= _W - 1)
             & (src_y >= 0) & (src_y <= _H - 1)).astype(jnp.float32)
    x0i = jnp.clip(x0, 0, _W - 1).astype(jnp.int32)
    x1i = jnp.clip(x0 + 1.0, 0, _W - 1).astype(jnp.int32)
    y0i = jnp.clip(y0, 0, _H - 1).astype(jnp.int32)
    y1i = jnp.clip(y0 + 1.0, 0, _H - 1).astype(jnp.int32)
    idx4 = jnp.stack([
        (y0i * _W + x0i).reshape(-1),
        (y0i * _W + x1i).reshape(-1),
        (y1i * _W + x0i).reshape(-1),
        (y1i * _W + x1i).reshape(-1),
    ])
    wt4 = jnp.stack([
        (wy0 * wx0 * valid).reshape(-1),
        (wy0 * wx1 * valid).reshape(-1),
        (wy1 * wx0 * valid).reshape(-1),
        (wy1 * wx1 * valid).reshape(-1),
    ])
    return idx4, wt4


def kernel(x, const):
    theta = jnp.squeeze(const, axis=0)[0]
    idx4, wt4 = _indices_weights(theta)
    table = x.reshape(_BC, _N).T
    out_t = _sc_interp(table, idx4, wt4)
    return out_t.T.reshape(x.shape)
